# Initial kernel scaffold; baseline (speedup 1.0000x reference)
import jax
import jax.numpy as jnp
from jax.experimental import pallas as pl
from jax.experimental.pallas import tpu as pltpu

N = 5850
E = 93600


def _leaky(v):
    return jnp.maximum(v, 0.01 * v)


def _gcn_agg(x, src, dst, norm, dinv2):
    s = jnp.zeros_like(x).at[dst].add(x[src] * norm[:, None])
    return s + dinv2[:, None] * x


_RB = 4500


def _fc2_body(x_ref, w_ref, b_ref, o_ref):
    i = pl.program_id(0)
    part = jnp.dot(x_ref[...], w_ref[...], preferred_element_type=jnp.float32)

    @pl.when(i == 0)
    def _():
        o_ref[...] = part + b_ref[...]

    @pl.when(i > 0)
    def _():
        o_ref[...] += part

    @pl.when(i == pl.num_programs(0) - 1)
    def _():
        o_ref[...] = jnp.maximum(o_ref[...], 0.01 * o_ref[...])


def _fc2(xf, W, b):
    out = pl.pallas_call(
        _fc2_body,
        grid=(13,),
        in_specs=[
            pl.BlockSpec((1, _RB), lambda i: (i, 0)),
            pl.BlockSpec((_RB, 100), lambda i: (i, 0)),
            pl.BlockSpec((1, 100), lambda i: (0, 0)),
        ],
        out_specs=pl.BlockSpec((1, 100), lambda i: (0, 0)),
        out_shape=jax.ShapeDtypeStruct((1, 100), jnp.float32),
    )(xf, W, b.reshape(1, 100))
    return out[0]


def kernel(pos, edge_index, W1, b1, W2, b2, fc1_W, fc1_b, fc2_W, fc2_b):
    src = edge_index[0]
    dst = edge_index[1]
    deg = jnp.ones((N,), jnp.float32).at[dst].add(1.0)
    dinv = deg ** -0.5
    norm = dinv[src] * dinv[dst]
    dinv2 = dinv * dinv

    x = _leaky(_gcn_agg(pos, src, dst, norm, dinv2) @ W1 + b1)
    x = _leaky(_gcn_agg(x, src, dst, norm, dinv2) @ W2 + b2)
    y = _leaky(x @ fc1_W + fc1_b)
    xf = y.reshape(13, 4500)
    return _fc2(xf, fc2_W, fc2_b)


# speedup vs baseline: 1.5933x; 1.5933x over previous
import jax
import jax.numpy as jnp
from jax.experimental import pallas as pl
from jax.experimental.pallas import tpu as pltpu

N = 5850
E = 93600


def _leaky(v):
    return jnp.maximum(v, 0.01 * v)


def _gcn_agg(x, src, dst, norm, dinv2):
    s = jnp.zeros_like(x).at[dst].add(x[src] * norm[:, None])
    return s + dinv2[:, None] * x


_RB = 4500


def _fc2_body(x_ref, w_ref, b_ref, o_ref):
    acc = b_ref[...]
    for i in range(13):
        acc = acc + jnp.dot(x_ref[i:i + 1, :], w_ref[_RB * i:_RB * (i + 1), :],
                            preferred_element_type=jnp.float32)
    o_ref[...] = jnp.maximum(acc, 0.01 * acc)


def _fc2(xf, W, b):
    out = pl.pallas_call(
        _fc2_body,
        out_shape=jax.ShapeDtypeStruct((1, 100), jnp.float32),
    )(xf, W, b.reshape(1, 100))
    return out[0]


def kernel(pos, edge_index, W1, b1, W2, b2, fc1_W, fc1_b, fc2_W, fc2_b):
    src = edge_index[0]
    dst = edge_index[1]
    deg = jnp.ones((N,), jnp.float32).at[dst].add(1.0)
    dinv = deg ** -0.5
    norm = dinv[src] * dinv[dst]
    dinv2 = dinv * dinv

    x = _leaky(_gcn_agg(pos, src, dst, norm, dinv2) @ W1 + b1)
    x = _leaky(_gcn_agg(x, src, dst, norm, dinv2) @ W2 + b2)
    y = _leaky(x @ fc1_W + fc1_b)
    xf = y.reshape(13, 4500)
    return _fc2(xf, fc2_W, fc2_b)


# speedup vs baseline: 24.7801x; 15.5525x over previous
import functools

import jax
import jax.numpy as jnp
from jax import lax
from jax.experimental import pallas as pl
from jax.experimental.pallas import tpu as pltpu
from jax.experimental.pallas import tpu_sc as plsc

N = 5850
E = 93600
NPAD = 5888
RNG = 368
EPT = 5856
EPT_LAST = 5760
F32 = jnp.float32

_mesh = plsc.VectorSubcoreMesh(core_axis_name="c", subcore_axis_name="s",
                               num_cores=1)


@functools.partial(
    pl.kernel,
    out_type=jax.ShapeDtypeStruct((58880,), F32),
    mesh=_mesh,
    compiler_params=pltpu.CompilerParams(needs_layout_passes=False),
    scratch_types=[
        pltpu.VMEM((17664,), F32),
        pltpu.VMEM((5888,), F32),
        pltpu.VMEM((29440,), F32),
        pltpu.VMEM((5856,), jnp.int32),
        pltpu.VMEM((5856,), jnp.int32),
        pltpu.VMEM((29440,), F32),
        pltpu.VMEM((352,), F32),
        pltpu.VMEM((3680,), F32),
        pltpu.VMEM_SHARED((52992,), F32),
    ],
)
def _sc_graph(pos_hbm, esrc_hbm, edst_hbm, smalls_hbm, y_hbm,
              posf_v, dinv_v, x1f_v, src_v, dst_v, msg_v, smalls_v, y_v,
              shared_s):
    t = lax.axis_index("s")
    lane = lax.iota(jnp.int32, 16)
    zero16 = jnp.zeros((16,), F32)

    def sget(idx):
        return smalls_v[pl.ds((idx // 16) * 16, 16)][idx % 16]

    pltpu.sync_copy(pos_hbm, posf_v.at[pl.ds(0, 17550)])
    pltpu.sync_copy(smalls_hbm, smalls_v)

    @pl.when(t < 15)
    def _():
        pltpu.sync_copy(esrc_hbm.at[pl.ds(t * EPT, EPT)], src_v)
        pltpu.sync_copy(edst_hbm.at[pl.ds(t * EPT, EPT)], dst_v)

    @pl.when(t == 15)
    def _():
        pltpu.sync_copy(esrc_hbm.at[pl.ds(15 * EPT, EPT_LAST)],
                        src_v.at[pl.ds(0, EPT_LAST)])
        pltpu.sync_copy(edst_hbm.at[pl.ds(15 * EPT, EPT_LAST)],
                        dst_v.at[pl.ds(0, EPT_LAST)])

    nvec = jnp.where(t == 15, EPT_LAST // 16, EPT // 16)

    def _zero_body(i, carry):
        dinv_v[pl.ds(i * 16, 16)] = zero16
        return carry
    lax.fori_loop(0, NPAD // 16, _zero_body, None)

    @pl.when(t < 9)
    def _():
        pltpu.sync_copy(dinv_v, shared_s.at[pl.ds(t * NPAD, NPAD)])

    plsc.subcore_barrier()

    def _ones_body(i, carry):
        msg_v[pl.ds(i * 16, 16)] = zero16 + 1.0
        return carry
    lax.fori_loop(0, EPT // 16, _ones_body, None)

    @pl.when(t < 15)
    def _():
        pltpu.sync_copy(msg_v.at[pl.ds(0, EPT)], shared_s.at[pl.ds(0, NPAD)].at[dst_v],
                        add=True)

    @pl.when(t == 15)
    def _():
        pltpu.sync_copy(msg_v.at[pl.ds(0, EPT_LAST)],
                        shared_s.at[pl.ds(0, NPAD)].at[dst_v.at[pl.ds(0, EPT_LAST)]],
                        add=True)

    plsc.subcore_barrier()

    pltpu.sync_copy(shared_s.at[pl.ds(0, NPAD)], dinv_v)

    def _dinv_body(i, carry):
        d = dinv_v[pl.ds(i * 16, 16)] + 1.0
        bits = lax.bitcast_convert_type(d, jnp.int32)
        y = lax.bitcast_convert_type(
            jnp.int32(0x5F3759DF) - lax.shift_right_logical(bits, 1), F32)
        for _ in range(3):
            y = y * (1.5 - 0.5 * d * y * y)
        dinv_v[pl.ds(i * 16, 16)] = y
        return carry
    lax.fori_loop(0, NPAD // 16, _dinv_body, None)

    def _m1_body(i, carry):
        sl = pl.ds(i * 16, 16)
        s16 = src_v[sl]
        d16 = dst_v[sl]
        nrm = plsc.load_gather(dinv_v, [s16]) * plsc.load_gather(dinv_v, [d16])
        base3 = s16 * 3
        for c in range(3):
            v = plsc.load_gather(posf_v, [base3 + c])
            msg_v[pl.ds(c * NPAD + i * 16, 16)] = v * nrm
        return carry
    lax.fori_loop(0, nvec, _m1_body, None)

    for c in range(3):
        @pl.when(t < 15)
        def _(c=c):
            pltpu.sync_copy(msg_v.at[pl.ds(c * NPAD, EPT)],
                            shared_s.at[pl.ds((1 + c) * NPAD, NPAD)].at[dst_v], add=True)

        @pl.when(t == 15)
        def _(c=c):
            pltpu.sync_copy(msg_v.at[pl.ds(c * NPAD, EPT_LAST)],
                            shared_s.at[pl.ds((1 + c) * NPAD, NPAD)].at[dst_v.at[pl.ds(0, EPT_LAST)]],
                            add=True)

    plsc.subcore_barrier()

    for c in range(3):
        pltpu.sync_copy(shared_s.at[pl.ds((1 + c) * NPAD, NPAD)],
                        msg_v.at[pl.ds(c * NPAD, NPAD)])

    def _x1_body(i, carry):
        sl = pl.ds(i * 16, 16)
        n16 = i * 16 + lane
        dv = dinv_v[sl]
        d2 = dv * dv
        ts = []
        for c in range(3):
            pc = plsc.load_gather(posf_v, [n16 * 3 + c])
            ts.append(msg_v[pl.ds(c * NPAD + i * 16, 16)] + d2 * pc)
        for j in range(5):
            acc = sget(15 + j) + ts[0] * sget(j) \
                + ts[1] * sget(5 + j) + ts[2] * sget(10 + j)
            x1f_v[pl.ds(j * NPAD + i * 16, 16)] = jnp.maximum(acc, 0.01 * acc)
        return carry
    lax.fori_loop(0, NPAD // 16, _x1_body, None)

    def _m2_body(i, carry):
        sl = pl.ds(i * 16, 16)
        s16 = src_v[sl]
        d16 = dst_v[sl]
        nrm = plsc.load_gather(dinv_v, [s16]) * plsc.load_gather(dinv_v, [d16])
        for c in range(5):
            v = plsc.load_gather(x1f_v, [s16 + c * NPAD])
            msg_v[pl.ds(c * NPAD + i * 16, 16)] = v * nrm
        return carry
    lax.fori_loop(0, nvec, _m2_body, None)

    for c in range(5):
        @pl.when(t < 15)
        def _(c=c):
            pltpu.sync_copy(msg_v.at[pl.ds(c * NPAD, EPT)],
                            shared_s.at[pl.ds((4 + c) * NPAD, NPAD)].at[dst_v], add=True)

        @pl.when(t == 15)
        def _(c=c):
            pltpu.sync_copy(msg_v.at[pl.ds(c * NPAD, EPT_LAST)],
                            shared_s.at[pl.ds((4 + c) * NPAD, NPAD)].at[dst_v.at[pl.ds(0, EPT_LAST)]],
                            add=True)

    plsc.subcore_barrier()

    base_n = t * RNG
    for c in range(5):
        pltpu.sync_copy(shared_s.at[pl.ds((4 + c) * NPAD + base_n, RNG)],
                        msg_v.at[pl.ds(c * NPAD, RNG)])

    def _fin_body(i, carry):
        dv = dinv_v[pl.ds(base_n + i * 16, 16)]
        d2 = dv * dv
        ts = []
        for c in range(5):
            xx = x1f_v[pl.ds(c * NPAD + base_n + i * 16, 16)]
            ts.append(msg_v[pl.ds(c * NPAD + i * 16, 16)] + d2 * xx)
        x2 = []
        for j in range(20):
            acc = sget(120 + j)
            for c in range(5):
                acc = acc + ts[c] * sget(20 + c * 20 + j)
            x2.append(jnp.maximum(acc, 0.01 * acc))
        nl10 = (i * 16 + lane) * 10
        for k in range(10):
            acc = sget(340 + k)
            for j in range(20):
                acc = acc + x2[j] * sget(140 + j * 10 + k)
            yv = jnp.maximum(acc, 0.01 * acc)
            plsc.store_scatter(y_v, [nl10 + k], yv)
        return carry
    lax.fori_loop(0, RNG // 16, _fin_body, None)

    pltpu.sync_copy(y_v, y_hbm.at[pl.ds(t * 3680, 3680)])


_RB = 4500


def _fc2_body(x_ref, w_ref, b_ref, o_ref):
    acc = b_ref[...]
    for i in range(13):
        acc = acc + jnp.dot(x_ref[i:i + 1, :], w_ref[_RB * i:_RB * (i + 1), :],
                            preferred_element_type=jnp.float32)
    o_ref[...] = jnp.maximum(acc, 0.01 * acc)


def _fc2(xf, W, b):
    out = pl.pallas_call(
        _fc2_body,
        out_shape=jax.ShapeDtypeStruct((1, 100), jnp.float32),
    )(xf, W, b.reshape(1, 100))
    return out[0]


def kernel(pos, edge_index, W1, b1, W2, b2, fc1_W, fc1_b, fc2_W, fc2_b):
    smalls = jnp.concatenate([
        W1.reshape(-1), b1, W2.reshape(-1), b2,
        fc1_W.reshape(-1), fc1_b, jnp.zeros((2,), jnp.float32)])
    y = _sc_graph(pos.reshape(-1), edge_index[0], edge_index[1], smalls)
    xf = y[:58500].reshape(13, 4500)
    return _fc2(xf, fc2_W, fc2_b)


# speedup vs baseline: 33.7564x; 1.3622x over previous
import functools

import jax
import jax.numpy as jnp
from jax import lax
from jax.experimental import pallas as pl
from jax.experimental.pallas import tpu as pltpu
from jax.experimental.pallas import tpu_sc as plsc

N = 5850
E = 93600
NPAD = 5888
RNG = 368
EPT = 5856
EPT_LAST = 5760
F32 = jnp.float32

R_DEG = 0
R_S1 = 1
R_S2 = 4
R_GP = 9
R_H = 12
NROWS = 17

_mesh = plsc.VectorSubcoreMesh(core_axis_name="c", subcore_axis_name="s",
                               num_cores=1)


@functools.partial(
    pl.kernel,
    out_type=jax.ShapeDtypeStruct((58880,), F32),
    mesh=_mesh,
    compiler_params=pltpu.CompilerParams(needs_layout_passes=False),
    scratch_types=[
        pltpu.VMEM((1104,), F32),
        pltpu.VMEM((368,), F32),
        pltpu.VMEM((1104,), F32),
        pltpu.VMEM((1104,), F32),
        pltpu.VMEM((1840,), F32),
        pltpu.VMEM((1840,), F32),
        pltpu.VMEM((5856,), jnp.int32),
        pltpu.VMEM((5856,), jnp.int32),
        pltpu.VMEM((29440,), F32),
        pltpu.VMEM((352,), F32),
        pltpu.VMEM((3680,), F32),
        pltpu.SemaphoreType.DMA,
        pltpu.VMEM_SHARED((NROWS * NPAD,), F32),
    ],
)
def _sc_graph(pos_hbm, esrc_hbm, edst_hbm, smalls_hbm, y_hbm,
              posr_v, dinv_v, gp_v, s1r_v, h_v, s2r_v, src_v, dst_v, msg_v,
              smalls_v, y_v, sem, shared_s):
    t = lax.axis_index("s")
    zero16 = jnp.zeros((16,), F32)
    base_n = t * RNG

    def sget(idx):
        return smalls_v[pl.ds((idx // 16) * 16, 16)][idx % 16]

    def srow(r, off, size):
        return shared_s.at[pl.ds(r * NPAD + off, size)]

    pltpu.sync_copy(smalls_hbm, smalls_v)

    @pl.when(t < 15)
    def _():
        pltpu.sync_copy(pos_hbm.at[pl.ds(t * 1104, 1104)], posr_v)
        pltpu.sync_copy(esrc_hbm.at[pl.ds(t * EPT, EPT)], src_v)
        pltpu.sync_copy(edst_hbm.at[pl.ds(t * EPT, EPT)], dst_v)

    @pl.when(t == 15)
    def _():
        pltpu.sync_copy(pos_hbm.at[pl.ds(15 * 1104, 990)],
                        posr_v.at[pl.ds(0, 990)])
        pltpu.sync_copy(esrc_hbm.at[pl.ds(15 * EPT, EPT_LAST)],
                        src_v.at[pl.ds(0, EPT_LAST)])
        pltpu.sync_copy(edst_hbm.at[pl.ds(15 * EPT, EPT_LAST)],
                        dst_v.at[pl.ds(0, EPT_LAST)])

    def _zero_body(i, carry):
        msg_v[pl.ds(i * 16, 16)] = zero16
        return carry
    lax.fori_loop(0, 9 * NPAD // (16 * 16), _zero_body, None)

    zchunk = 9 * NPAD // 16
    pltpu.sync_copy(msg_v.at[pl.ds(0, zchunk)],
                    shared_s.at[pl.ds(t * zchunk, zchunk)])

    plsc.subcore_barrier()

    def _ones_body(i, carry):
        msg_v[pl.ds(i * 16, 16)] = zero16 + 1.0
        return carry
    lax.fori_loop(0, EPT // 16, _ones_body, None)

    @pl.when(t < 15)
    def _():
        pltpu.sync_copy(msg_v.at[pl.ds(0, EPT)], srow(R_DEG, 0, NPAD).at[dst_v],
                        add=True)

    @pl.when(t == 15)
    def _():
        pltpu.sync_copy(msg_v.at[pl.ds(0, EPT_LAST)],
                        srow(R_DEG, 0, NPAD).at[dst_v.at[pl.ds(0, EPT_LAST)]],
                        add=True)

    plsc.subcore_barrier()

    pltpu.sync_copy(srow(R_DEG, base_n, RNG), dinv_v)

    lane = lax.iota(jnp.int32, 16)

    def _dinv_body(i, carry):
        d = dinv_v[pl.ds(i * 16, 16)] + 1.0
        bits = lax.bitcast_convert_type(d, jnp.int32)
        y = lax.bitcast_convert_type(
            jnp.int32(0x5F3759DF) - lax.shift_right_logical(bits, 1), F32)
        for _ in range(3):
            y = y * (1.5 - 0.5 * d * y * y)
        dinv_v[pl.ds(i * 16, 16)] = y
        nloc3 = (i * 16 + lane) * 3
        for c in range(3):
            pc = plsc.load_gather(posr_v, [nloc3 + c])
            gp_v[pl.ds(c * RNG + i * 16, 16)] = y * pc
        return carry
    lax.fori_loop(0, RNG // 16, _dinv_body, None)

    for c in range(3):
        pltpu.sync_copy(gp_v.at[pl.ds(c * RNG, RNG)],
                        srow(R_GP + c, base_n, RNG))

    plsc.subcore_barrier()

    def _msg_pass(rows_from, rows_to, nfeat):
        waits = []
        for c in range(nfeat):
            mrow = msg_v.at[pl.ds(c * NPAD, EPT)]
            mrow_l = msg_v.at[pl.ds(c * NPAD, EPT_LAST)]

            @pl.when(t < 15)
            def _(c=c, mrow=mrow):
                pltpu.sync_copy(srow(rows_from + c, 0, NPAD).at[src_v], mrow)

            @pl.when(t == 15)
            def _(c=c, mrow_l=mrow_l):
                pltpu.sync_copy(
                    srow(rows_from + c, 0, NPAD).at[src_v.at[pl.ds(0, EPT_LAST)]],
                    mrow_l)

            @pl.when(t < 15)
            def _(c=c, mrow=mrow):
                pltpu.async_copy(mrow, srow(rows_to + c, 0, NPAD).at[dst_v],
                                 sem, add=True)

            @pl.when(t == 15)
            def _(c=c, mrow_l=mrow_l):
                pltpu.async_copy(
                    mrow_l,
                    srow(rows_to + c, 0, NPAD).at[dst_v.at[pl.ds(0, EPT_LAST)]],
                    sem, add=True)

        for c in range(nfeat):
            @pl.when(t < 15)
            def _(c=c):
                pltpu.make_async_copy(
                    msg_v.at[pl.ds(c * NPAD, EPT)],
                    srow(rows_to + c, 0, NPAD).at[dst_v], sem).wait()

            @pl.when(t == 15)
            def _(c=c):
                pltpu.make_async_copy(
                    msg_v.at[pl.ds(c * NPAD, EPT_LAST)],
                    srow(rows_to + c, 0, NPAD).at[dst_v.at[pl.ds(0, EPT_LAST)]],
                    sem).wait()

    _msg_pass(R_GP, R_S1, 3)
    plsc.subcore_barrier()

    for c in range(3):
        pltpu.sync_copy(srow(R_S1 + c, base_n, RNG),
                        s1r_v.at[pl.ds(c * RNG, RNG)])

    def _x1_body(i, carry):
        dv = dinv_v[pl.ds(i * 16, 16)]
        ts = []
        for c in range(3):
            ts.append(dv * (s1r_v[pl.ds(c * RNG + i * 16, 16)]
                            + gp_v[pl.ds(c * RNG + i * 16, 16)]))
        for j in range(5):
            acc = sget(15 + j) + ts[0] * sget(j) \
                + ts[1] * sget(5 + j) + ts[2] * sget(10 + j)
            h_v[pl.ds(j * RNG + i * 16, 16)] = dv * jnp.maximum(acc, 0.01 * acc)
        return carry
    lax.fori_loop(0, RNG // 16, _x1_body, None)

    for c in range(5):
        pltpu.sync_copy(h_v.at[pl.ds(c * RNG, RNG)],
                        srow(R_H + c, base_n, RNG))

    plsc.subcore_barrier()

    _msg_pass(R_H, R_S2, 5)
    plsc.subcore_barrier()

    for c in range(5):
        pltpu.sync_copy(srow(R_S2 + c, base_n, RNG),
                        s2r_v.at[pl.ds(c * RNG, RNG)])

    def _fin_body(i, carry):
        dv = dinv_v[pl.ds(i * 16, 16)]
        ts = []
        for c in range(5):
            ts.append(dv * (s2r_v[pl.ds(c * RNG + i * 16, 16)]
                            + h_v[pl.ds(c * RNG + i * 16, 16)]))
        x2 = []
        for j in range(20):
            acc = sget(120 + j)
            for c in range(5):
                acc = acc + ts[c] * sget(20 + c * 20 + j)
            x2.append(jnp.maximum(acc, 0.01 * acc))
        nl10 = (i * 16 + lane) * 10
        for k in range(10):
            acc = sget(340 + k)
            for j in range(20):
                acc = acc + x2[j] * sget(140 + j * 10 + k)
            yv = jnp.maximum(acc, 0.01 * acc)
            plsc.store_scatter(y_v, [nl10 + k], yv)
        return carry
    lax.fori_loop(0, RNG // 16, _fin_body, None)

    pltpu.sync_copy(y_v, y_hbm.at[pl.ds(t * 3680, 3680)])


_RB = 4500


def _fc2_body(x_ref, w_ref, b_ref, o_ref):
    acc = b_ref[...]
    for i in range(13):
        acc = acc + jnp.dot(x_ref[i:i + 1, :], w_ref[_RB * i:_RB * (i + 1), :],
                            preferred_element_type=jnp.float32)
    o_ref[...] = jnp.maximum(acc, 0.01 * acc)


def _fc2(xf, W, b):
    out = pl.pallas_call(
        _fc2_body,
        out_shape=jax.ShapeDtypeStruct((1, 100), jnp.float32),
    )(xf, W, b.reshape(1, 100))
    return out[0]


def kernel(pos, edge_index, W1, b1, W2, b2, fc1_W, fc1_b, fc2_W, fc2_b):
    smalls = jnp.concatenate([
        W1.reshape(-1), b1, W2.reshape(-1), b2,
        fc1_W.reshape(-1), fc1_b, jnp.zeros((2,), jnp.float32)])
    y = _sc_graph(pos.reshape(-1), edge_index[0], edge_index[1], smalls)
    xf = y[:58500].reshape(13, 4500)
    return _fc2(xf, fc2_W, fc2_b)


# speedup vs baseline: 33.8480x; 1.0027x over previous
import functools

import jax
import jax.numpy as jnp
from jax import lax
from jax.experimental import pallas as pl
from jax.experimental.pallas import tpu as pltpu
from jax.experimental.pallas import tpu_sc as plsc

N = 5850
E = 93600
NPAD = 5888
RNG = 368
EPT = 5856
EPT_LAST = 5760
F32 = jnp.float32

R_DEG = 0
R_S1 = 1
R_S2 = 4
R_GP = 9
R_H = 12
NROWS = 17

_mesh = plsc.VectorSubcoreMesh(core_axis_name="c", subcore_axis_name="s",
                               num_cores=1)


@functools.partial(
    pl.kernel,
    out_type=jax.ShapeDtypeStruct((58880,), F32),
    mesh=_mesh,
    compiler_params=pltpu.CompilerParams(needs_layout_passes=False),
    scratch_types=[
        pltpu.VMEM((1104,), F32),
        pltpu.VMEM((368,), F32),
        pltpu.VMEM((1104,), F32),
        pltpu.VMEM((1104,), F32),
        pltpu.VMEM((1840,), F32),
        pltpu.VMEM((1840,), F32),
        pltpu.VMEM((5856,), jnp.int32),
        pltpu.VMEM((5856,), jnp.int32),
        pltpu.VMEM((29440,), F32),
        pltpu.VMEM((352,), F32),
        pltpu.VMEM((3680,), F32),
        pltpu.SemaphoreType.DMA,
        pltpu.VMEM_SHARED((NROWS * NPAD,), F32),
    ],
)
def _sc_graph(pos_hbm, esrc_hbm, edst_hbm, smalls_hbm, y_hbm,
              posr_v, dinv_v, gp_v, s1r_v, h_v, s2r_v, src_v, dst_v, msg_v,
              smalls_v, y_v, sem, shared_s):
    t = lax.axis_index("s")
    zero16 = jnp.zeros((16,), F32)
    base_n = t * RNG

    def sget(idx):
        return smalls_v[pl.ds((idx // 16) * 16, 16)][idx % 16]

    def srow(r, off, size):
        return shared_s.at[pl.ds(r * NPAD + off, size)]

    pltpu.sync_copy(smalls_hbm, smalls_v)

    @pl.when(t < 15)
    def _():
        pltpu.sync_copy(pos_hbm.at[pl.ds(t * 1104, 1104)], posr_v)
        pltpu.sync_copy(esrc_hbm.at[pl.ds(t * EPT, EPT)], src_v)
        pltpu.sync_copy(edst_hbm.at[pl.ds(t * EPT, EPT)], dst_v)

    @pl.when(t == 15)
    def _():
        pltpu.sync_copy(pos_hbm.at[pl.ds(15 * 1104, 990)],
                        posr_v.at[pl.ds(0, 990)])
        pltpu.sync_copy(esrc_hbm.at[pl.ds(15 * EPT, EPT_LAST)],
                        src_v.at[pl.ds(0, EPT_LAST)])
        pltpu.sync_copy(edst_hbm.at[pl.ds(15 * EPT, EPT_LAST)],
                        dst_v.at[pl.ds(0, EPT_LAST)])

    def _zero_body(i, carry):
        msg_v[pl.ds(i * 16, 16)] = zero16
        return carry
    lax.fori_loop(0, 9 * NPAD // (16 * 16), _zero_body, None)

    zchunk = 9 * NPAD // 16
    pltpu.sync_copy(msg_v.at[pl.ds(0, zchunk)],
                    shared_s.at[pl.ds(t * zchunk, zchunk)])

    plsc.subcore_barrier()

    def _ones_body(i, carry):
        msg_v[pl.ds(i * 16, 16)] = zero16 + 1.0
        return carry
    lax.fori_loop(0, EPT // 16, _ones_body, None)

    @pl.when(t < 15)
    def _():
        pltpu.sync_copy(msg_v.at[pl.ds(0, EPT)], srow(R_DEG, 0, NPAD).at[dst_v],
                        add=True)

    @pl.when(t == 15)
    def _():
        pltpu.sync_copy(msg_v.at[pl.ds(0, EPT_LAST)],
                        srow(R_DEG, 0, NPAD).at[dst_v.at[pl.ds(0, EPT_LAST)]],
                        add=True)

    plsc.subcore_barrier()

    pltpu.sync_copy(srow(R_DEG, base_n, RNG), dinv_v)

    lane = lax.iota(jnp.int32, 16)

    def _dinv_body(i, carry):
        d = dinv_v[pl.ds(i * 16, 16)] + 1.0
        bits = lax.bitcast_convert_type(d, jnp.int32)
        y = lax.bitcast_convert_type(
            jnp.int32(0x5F3759DF) - lax.shift_right_logical(bits, 1), F32)
        for _ in range(3):
            y = y * (1.5 - 0.5 * d * y * y)
        dinv_v[pl.ds(i * 16, 16)] = y
        nloc3 = (i * 16 + lane) * 3
        for c in range(3):
            pc = plsc.load_gather(posr_v, [nloc3 + c])
            gp_v[pl.ds(c * RNG + i * 16, 16)] = y * pc
        return carry
    lax.fori_loop(0, RNG // 16, _dinv_body, None)

    for c in range(3):
        pltpu.sync_copy(gp_v.at[pl.ds(c * RNG, RNG)],
                        srow(R_GP + c, base_n, RNG))

    plsc.subcore_barrier()

    def _msg_pass(rows_from, rows_to, nfeat):
        def gsrc(c):
            return srow(rows_from + c, 0, NPAD)

        def sdst(c, last):
            idx = dst_v.at[pl.ds(0, EPT_LAST)] if last else dst_v
            return srow(rows_to + c, 0, NPAD).at[idx]

        def mrow(c, last):
            return msg_v.at[pl.ds(c * NPAD, EPT_LAST if last else EPT)]

        @pl.when(t < 15)
        def _():
            for c in range(nfeat):
                pltpu.async_copy(gsrc(c).at[src_v], mrow(c, False), sem)
            for c in range(nfeat):
                pltpu.make_async_copy(gsrc(c).at[src_v], mrow(c, False),
                                      sem).wait()
            for c in range(nfeat):
                pltpu.async_copy(mrow(c, False), sdst(c, False), sem, add=True)
            for c in range(nfeat):
                pltpu.make_async_copy(mrow(c, False), sdst(c, False),
                                      sem).wait()

        @pl.when(t == 15)
        def _():
            srcl = src_v.at[pl.ds(0, EPT_LAST)]
            for c in range(nfeat):
                pltpu.async_copy(gsrc(c).at[srcl], mrow(c, True), sem)
            for c in range(nfeat):
                pltpu.make_async_copy(gsrc(c).at[srcl], mrow(c, True),
                                      sem).wait()
            for c in range(nfeat):
                pltpu.async_copy(mrow(c, True), sdst(c, True), sem, add=True)
            for c in range(nfeat):
                pltpu.make_async_copy(mrow(c, True), sdst(c, True), sem).wait()

    _msg_pass(R_GP, R_S1, 3)
    plsc.subcore_barrier()

    for c in range(3):
        pltpu.sync_copy(srow(R_S1 + c, base_n, RNG),
                        s1r_v.at[pl.ds(c * RNG, RNG)])

    def _x1_body(i, carry):
        dv = dinv_v[pl.ds(i * 16, 16)]
        ts = []
        for c in range(3):
            ts.append(dv * (s1r_v[pl.ds(c * RNG + i * 16, 16)]
                            + gp_v[pl.ds(c * RNG + i * 16, 16)]))
        for j in range(5):
            acc = sget(15 + j) + ts[0] * sget(j) \
                + ts[1] * sget(5 + j) + ts[2] * sget(10 + j)
            h_v[pl.ds(j * RNG + i * 16, 16)] = dv * jnp.maximum(acc, 0.01 * acc)
        return carry
    lax.fori_loop(0, RNG // 16, _x1_body, None)

    for c in range(5):
        pltpu.sync_copy(h_v.at[pl.ds(c * RNG, RNG)],
                        srow(R_H + c, base_n, RNG))

    plsc.subcore_barrier()

    _msg_pass(R_H, R_S2, 5)
    plsc.subcore_barrier()

    for c in range(5):
        pltpu.sync_copy(srow(R_S2 + c, base_n, RNG),
                        s2r_v.at[pl.ds(c * RNG, RNG)])

    def _fin_body(i, carry):
        dv = dinv_v[pl.ds(i * 16, 16)]
        ts = []
        for c in range(5):
            ts.append(dv * (s2r_v[pl.ds(c * RNG + i * 16, 16)]
                            + h_v[pl.ds(c * RNG + i * 16, 16)]))
        x2 = []
        for j in range(20):
            acc = sget(120 + j)
            for c in range(5):
                acc = acc + ts[c] * sget(20 + c * 20 + j)
            x2.append(jnp.maximum(acc, 0.01 * acc))
        nl10 = (i * 16 + lane) * 10
        for k in range(10):
            acc = sget(340 + k)
            for j in range(20):
                acc = acc + x2[j] * sget(140 + j * 10 + k)
            yv = jnp.maximum(acc, 0.01 * acc)
            plsc.store_scatter(y_v, [nl10 + k], yv)
        return carry
    lax.fori_loop(0, RNG // 16, _fin_body, None)

    pltpu.sync_copy(y_v, y_hbm.at[pl.ds(t * 3680, 3680)])


_RB = 4500


def _fc2_body(x_ref, w_ref, b_ref, o_ref):
    acc = b_ref[...]
    for i in range(13):
        acc = acc + jnp.dot(x_ref[i:i + 1, :], w_ref[_RB * i:_RB * (i + 1), :],
                            preferred_element_type=jnp.float32)
    o_ref[...] = jnp.maximum(acc, 0.01 * acc)


def _fc2(xf, W, b):
    out = pl.pallas_call(
        _fc2_body,
        out_shape=jax.ShapeDtypeStruct((1, 100), jnp.float32),
    )(xf, W, b.reshape(1, 100))
    return out[0]


def kernel(pos, edge_index, W1, b1, W2, b2, fc1_W, fc1_b, fc2_W, fc2_b):
    smalls = jnp.concatenate([
        W1.reshape(-1), b1, W2.reshape(-1), b2,
        fc1_W.reshape(-1), fc1_b, jnp.zeros((2,), jnp.float32)])
    y = _sc_graph(pos.reshape(-1), edge_index[0], edge_index[1], smalls)
    xf = y[:58500].reshape(13, 4500)
    return _fc2(xf, fc2_W, fc2_b)
